# baseline (speedup 1.0000x reference)
import jax
import jax.numpy as jnp
from jax.experimental import pallas as pl
from jax.experimental.pallas import tpu as pltpu

_N = 8192
_R = 64
_L = 128
_U = 64


_G = 2
_RB = _R // _G


def _vq_kernel(w_ref, x_ref, val_ref, idx_ref):
    xv = x_ref[...]

    def body(t, carry):
        bd, bj, bv = carry
        for u in range(_U):
            j = t * _U + u
            c = w_ref[j]
            d = xv - c
            d = d * d
            m = d < bd
            bd = jnp.where(m, d, bd)
            bj = jnp.where(m, j, bj)
            bv = jnp.where(m, c, bv)
        return bd, bj, bv

    bd0 = jnp.full((_RB, _L), jnp.inf, jnp.float32)
    bj0 = jnp.zeros((_RB, _L), jnp.int32)
    bv0 = jnp.zeros((_RB, _L), jnp.float32)
    _, bj, bv = jax.lax.fori_loop(0, _N // _U, body, (bd0, bj0, bv0))

    idx_ref[...] = bj
    val_ref[...] = bv


def kernel(x, weight):
    shape = x.shape
    xf = x.reshape(_R, _L)
    wf = weight.reshape(_N)
    val, idx = pl.pallas_call(
        _vq_kernel,
        grid=(_G,),
        in_specs=[
            pl.BlockSpec(memory_space=pltpu.MemorySpace.SMEM),
            pl.BlockSpec((_RB, _L), lambda i: (i, 0)),
        ],
        out_specs=[
            pl.BlockSpec((_RB, _L), lambda i: (i, 0)),
            pl.BlockSpec((_RB, _L), lambda i: (i, 0)),
        ],
        out_shape=[
            jax.ShapeDtypeStruct((_R, _L), jnp.float32),
            jax.ShapeDtypeStruct((_R, _L), jnp.int32),
        ],
        compiler_params=pltpu.CompilerParams(
            dimension_semantics=("parallel",),
        ),
    )(wf, xf)
    return val.reshape(shape), idx.reshape(shape)
